# Initial kernel scaffold; baseline (speedup 1.0000x reference)
import functools

import numpy as np
import jax
import jax.numpy as jnp
from jax import lax
from jax.experimental import pallas as pl
from jax.experimental.pallas import tpu as pltpu
from jax.experimental.pallas import tpu_sc as plsc

N = 10000
E = 320000
G = 64
D = 128

NPAD = 10240
NB = 1024
NBLK = NPAD // NB

CORES = 2
SUBC = 16
C = 128
EPW = 20096
EPAD = EPW * SUBC
NCHUNK = EPW // C
RPS = NPAD // SUBC

ROWA = 16
ROWH = 64
ROWACC = 80

_NEG = -1e30

_KRED8 = np.kron(np.eye(8), np.ones((16, 1))).astype(np.float32)
_EYE8_16 = np.eye(8, 16).astype(np.float32)
_EYE1_16 = np.eye(1, 16).astype(np.float32)
_BC4_64 = np.kron(np.eye(4), np.ones((1, 16))).astype(np.float32)
_ONES128_1 = np.ones((128, 1), np.float32)


def _leaky(x):
    return jnp.where(x >= 0, x, x * jnp.float32(0.2))


def _tc1_body(x_ref, w_ref, ats_ref, atd_ref,
              hs_ref, as_ref, ad_ref, acc_ref):
    i = pl.program_id(0)
    h = jnp.dot(x_ref[...], w_ref[...], preferred_element_type=jnp.float32)
    a_s = jnp.dot(h * ats_ref[...], _KRED8)
    a_d = jnp.dot(h * atd_ref[...], _KRED8)
    rows = i * NB + lax.broadcasted_iota(jnp.int32, (NB, 1), 0)
    mask = rows < N
    as_ref[...] = jnp.dot(jnp.where(mask, a_s, _NEG), _EYE8_16)
    ad_ref[...] = jnp.dot(jnp.where(mask, a_d, 0.0), _EYE8_16)
    ex_self = jnp.exp(_leaky(a_s + a_d))
    den8 = jnp.where(mask, ex_self, 1.0)
    for c in range(2):
        hc = h[:, 64 * c:64 * c + 64]
        hs_ref[c] = hc
        acc_ref[c, :, 0:64] = hc * jnp.dot(ex_self[:, 4 * c:4 * c + 4], _BC4_64)
        acc_ref[c, :, 64:68] = den8[:, 4 * c:4 * c + 4]
        acc_ref[c, :, 68:80] = jnp.zeros((NB, 12), jnp.float32)


_tc1 = pl.pallas_call(
    _tc1_body,
    grid=(NBLK,),
    in_specs=[
        pl.BlockSpec((NB, D), lambda i: (i, 0)),
        pl.BlockSpec((D, D), lambda i: (0, 0)),
        pl.BlockSpec((1, D), lambda i: (0, 0)),
        pl.BlockSpec((1, D), lambda i: (0, 0)),
    ],
    out_specs=[
        pl.BlockSpec((2, NB, 64), lambda i: (0, i, 0)),
        pl.BlockSpec((NB, ROWA), lambda i: (i, 0)),
        pl.BlockSpec((NB, ROWA), lambda i: (i, 0)),
        pl.BlockSpec((2, NB, ROWACC), lambda i: (0, i, 0)),
    ],
    out_shape=[
        jax.ShapeDtypeStruct((2, NPAD, 64), jnp.float32),
        jax.ShapeDtypeStruct((NPAD, ROWA), jnp.float32),
        jax.ShapeDtypeStruct((NPAD, ROWA), jnp.float32),
        jax.ShapeDtypeStruct((2, NPAD, ROWACC), jnp.float32),
    ],
)


def _tc2_body(acc_ref, w2_ref, b1_ref, ats2_ref, atd2_ref,
              hs2_ref, as2_ref, ad2_ref, acc2_ref):
    i = pl.program_id(0)
    parts = []
    for c in range(2):
        num = acc_ref[c, :, 0:64]
        den = acc_ref[c, :, 64:68]
        parts.append(num / jnp.dot(den, _BC4_64))
    out1 = jnp.concatenate(parts, axis=1) + b1_ref[...]
    hmid = jnp.where(out1 > 0, out1, jnp.exp(jnp.minimum(out1, 0.0)) - 1.0)
    h2 = jnp.dot(hmid, w2_ref[...], preferred_element_type=jnp.float32)
    a_s = jnp.dot(h2 * ats2_ref[...], _ONES128_1)
    a_d = jnp.dot(h2 * atd2_ref[...], _ONES128_1)
    rows = i * NB + lax.broadcasted_iota(jnp.int32, (NB, 1), 0)
    mask = rows < N
    as2_ref[...] = jnp.dot(jnp.where(mask, a_s, _NEG), _EYE1_16)
    ad2_ref[...] = jnp.dot(jnp.where(mask, a_d, 0.0), _EYE1_16)
    ex_self = jnp.exp(_leaky(a_s + a_d))
    den1 = jnp.where(mask, ex_self, 1.0)
    for c in range(2):
        h2c = h2[:, 64 * c:64 * c + 64]
        hs2_ref[c] = h2c
        acc2_ref[c, :, 0:64] = h2c * ex_self
        acc2_ref[c, :, 64:65] = den1
        acc2_ref[c, :, 65:80] = jnp.zeros((NB, 15), jnp.float32)


_tc2 = pl.pallas_call(
    _tc2_body,
    grid=(NBLK,),
    in_specs=[
        pl.BlockSpec((2, NB, ROWACC), lambda i: (0, i, 0)),
        pl.BlockSpec((D, D), lambda i: (0, 0)),
        pl.BlockSpec((1, D), lambda i: (0, 0)),
        pl.BlockSpec((1, D), lambda i: (0, 0)),
        pl.BlockSpec((1, D), lambda i: (0, 0)),
    ],
    out_specs=[
        pl.BlockSpec((2, NB, 64), lambda i: (0, i, 0)),
        pl.BlockSpec((NB, ROWA), lambda i: (i, 0)),
        pl.BlockSpec((NB, ROWA), lambda i: (i, 0)),
        pl.BlockSpec((2, NB, ROWACC), lambda i: (0, i, 0)),
    ],
    out_shape=[
        jax.ShapeDtypeStruct((2, NPAD, 64), jnp.float32),
        jax.ShapeDtypeStruct((NPAD, ROWA), jnp.float32),
        jax.ShapeDtypeStruct((NPAD, ROWA), jnp.float32),
        jax.ShapeDtypeStruct((2, NPAD, ROWACC), jnp.float32),
    ],
)


def _tc3_body(acc_ref, b2_ref, batch_ref, out_ref):
    i = pl.program_id(0)
    parts = []
    for c in range(2):
        parts.append(acc_ref[c, :, 0:64] / acc_ref[c, :, 64:65])
    out2 = jnp.concatenate(parts, axis=1) + b2_ref[...]
    oh = (batch_ref[...] == lax.broadcasted_iota(jnp.int32, (NB, G), 1))
    contrib = lax.dot_general(oh.astype(jnp.float32), out2,
                              (((0,), (0,)), ((), ())),
                              preferred_element_type=jnp.float32)

    @pl.when(i == 0)
    def _():
        out_ref[...] = contrib

    @pl.when(i > 0)
    def _():
        out_ref[...] += contrib


_tc3 = pl.pallas_call(
    _tc3_body,
    grid=(NBLK,),
    in_specs=[
        pl.BlockSpec((2, NB, ROWACC), lambda i: (0, i, 0)),
        pl.BlockSpec((1, D), lambda i: (0, 0)),
        pl.BlockSpec((NB, 1), lambda i: (i, 0)),
    ],
    out_specs=pl.BlockSpec((G, D), lambda i: (0, 0)),
    out_shape=jax.ShapeDtypeStruct((G, D), jnp.float32),
)


def _take16(v, idx):
    dnums = lax.GatherDimensionNumbers(
        offset_dims=(), collapsed_slice_dims=(0,), start_index_map=(0,))
    return lax.gather(v, idx[:, None], dnums, slice_sizes=(1,),
                      mode=lax.GatherScatterMode.PROMISE_IN_BOUNDS)


def _make_sc_edge(hpc):
    mesh = plsc.VectorSubcoreMesh(core_axis_name="c", subcore_axis_name="s")

    @functools.partial(
        pl.kernel,
        out_type=jax.ShapeDtypeStruct((CORES, NPAD, ROWACC), jnp.float32),
        mesh=mesh,
        scratch_types=[
            pltpu.VMEM((C,), jnp.int32),
            pltpu.VMEM((C,), jnp.int32),
            pltpu.VMEM((C, ROWA), jnp.float32),
            pltpu.VMEM((C, ROWA), jnp.float32),
            pltpu.VMEM((C, ROWH), jnp.float32),
            pltpu.VMEM((C, ROWACC), jnp.float32),
            pltpu.VMEM_SHARED((NPAD, ROWA), jnp.float32),
            pltpu.VMEM_SHARED((NPAD, ROWA), jnp.float32),
            pltpu.VMEM_SHARED((NPAD, ROWH), jnp.float32),
            pltpu.VMEM_SHARED((NPAD, ROWACC), jnp.float32),
        ],
    )
    def sc_edge(src_hbm, dst_hbm, astab_hbm, adtab_hbm, hsplit_hbm, accinit_hbm,
                out_hbm, srcv, dstv, avs, avd, hv, mv, as_sh, ad_sh, h_sh, acc_sh):
        cid = lax.axis_index("c")
        sid = lax.axis_index("s")
        r0 = sid * RPS
        pltpu.sync_copy(astab_hbm.at[pl.ds(r0, RPS)], as_sh.at[pl.ds(r0, RPS)])
        pltpu.sync_copy(adtab_hbm.at[pl.ds(r0, RPS)], ad_sh.at[pl.ds(r0, RPS)])
        pltpu.sync_copy(hsplit_hbm.at[cid, pl.ds(r0, RPS)], h_sh.at[pl.ds(r0, RPS)])
        pltpu.sync_copy(accinit_hbm.at[cid, pl.ds(r0, RPS)], acc_sh.at[pl.ds(r0, RPS)])
        plsc.subcore_barrier()

        iota16 = lax.iota(jnp.int32, 16)
        lane0 = cid * 4 if hpc == 4 else 0
        tail_idx = jnp.minimum(iota16, hpc - 1) + lane0
        tail_mask = iota16 < hpc
        ebase = sid * EPW

        def chunk_body(i, _):
            off = ebase + i * C
            pltpu.sync_copy(src_hbm.at[pl.ds(off, C)], srcv)
            pltpu.sync_copy(dst_hbm.at[pl.ds(off, C)], dstv)
            pltpu.sync_copy(as_sh.at[srcv], avs)
            pltpu.sync_copy(ad_sh.at[dstv], avd)
            pltpu.sync_copy(h_sh.at[srcv], hv)

            def edge_body(e, _):
                ex = jnp.exp(_leaky(avs[e] + avd[e]))
                if hpc == 1:
                    exb0 = _take16(ex, jnp.zeros((16,), jnp.int32))
                for j in range(4):
                    if hpc == 4:
                        exb = _take16(ex, jnp.full((16,), j, jnp.int32) + lane0)
                    else:
                        exb = exb0
                    mv[e, pl.ds(16 * j, 16)] = hv[e, pl.ds(16 * j, 16)] * exb
                tail = jnp.where(tail_mask, _take16(ex, tail_idx), 0.0)
                mv[e, pl.ds(64, 16)] = tail
                return 0

            lax.fori_loop(0, C, edge_body, 0)
            pltpu.sync_copy(mv, acc_sh.at[dstv], add=True)
            return 0

        lax.fori_loop(0, NCHUNK, chunk_body, 0)
        plsc.subcore_barrier()
        pltpu.sync_copy(acc_sh.at[pl.ds(r0, RPS)],
                        out_hbm.at[cid, pl.ds(r0, RPS)])

    return sc_edge


_sc_edge_l1 = _make_sc_edge(4)
_sc_edge_l2 = _make_sc_edge(1)


def kernel(x, edge_index, batch, W1, att_src1, att_dst1, b1,
           W2, att_src2, att_dst2, b2):
    x_p = jnp.pad(x, ((0, NPAD - N), (0, 0)))
    pad_idx = N + (jnp.arange(EPAD - E, dtype=jnp.int32) % (NPAD - N))
    src = jnp.concatenate([edge_index[0], pad_idx])
    dst = jnp.concatenate([edge_index[1], pad_idx])
    batch_p = jnp.pad(batch, (0, NPAD - N), constant_values=G).reshape(NPAD, 1)

    hs, as1, ad1, acc1_init = _tc1(x_p, W1,
                                   att_src1.reshape(1, D),
                                   att_dst1.reshape(1, D))
    acc1 = _sc_edge_l1(src, dst, as1, ad1, hs, acc1_init)
    hs2, as2, ad2, acc2_init = _tc2(acc1, W2, b1.reshape(1, D),
                                    att_src2.reshape(1, D),
                                    att_dst2.reshape(1, D))
    acc2 = _sc_edge_l2(src, dst, as2, ad2, hs2, acc2_init)
    return _tc3(acc2, b2.reshape(1, D), batch_p)


# speedup vs baseline: 20.6494x; 20.6494x over previous
import functools

import numpy as np
import jax
import jax.numpy as jnp
from jax import lax
from jax.experimental import pallas as pl
from jax.experimental.pallas import tpu as pltpu
from jax.experimental.pallas import tpu_sc as plsc

N = 10000
E = 320000
G = 64
D = 128

NPAD = 10240
NB = 1024
NBLK = NPAD // NB

CORES = 2
SUBC = 16
C = 128
EPW = 20096
EPAD = EPW * SUBC
NCHUNK = EPW // C
RPS = NPAD // SUBC

ROWA = 16
ROWH = 64
ROWACC = 80

_NEG = -1e30

def _kred8():
    r = lax.broadcasted_iota(jnp.int32, (D, 8), 0)
    c = lax.broadcasted_iota(jnp.int32, (D, 8), 1)
    return (r // 16 == c).astype(jnp.float32)


def _eye(m, n):
    r = lax.broadcasted_iota(jnp.int32, (m, n), 0)
    c = lax.broadcasted_iota(jnp.int32, (m, n), 1)
    return (r == c).astype(jnp.float32)


def _bc4_64():
    r = lax.broadcasted_iota(jnp.int32, (4, 64), 0)
    c = lax.broadcasted_iota(jnp.int32, (4, 64), 1)
    return (c // 16 == r).astype(jnp.float32)


def _leaky(x):
    return jnp.where(x >= 0, x, x * jnp.float32(0.2))


def _tc1_body(x_ref, w_ref, ats_ref, atd_ref,
              hs_ref, as_ref, ad_ref, acc_ref):
    i = pl.program_id(0)
    h = jnp.dot(x_ref[...], w_ref[...], preferred_element_type=jnp.float32)
    a_s = jnp.dot(h * ats_ref[...], _kred8())
    a_d = jnp.dot(h * atd_ref[...], _kred8())
    rows = i * NB + lax.broadcasted_iota(jnp.int32, (NB, 1), 0)
    mask = rows < N
    as_ref[...] = jnp.dot(jnp.where(mask, a_s, _NEG), _eye(8, 16))
    ad_ref[...] = jnp.dot(jnp.where(mask, a_d, 0.0), _eye(8, 16))
    ex_self = jnp.exp(_leaky(a_s + a_d))
    den8 = jnp.where(mask, ex_self, 1.0)
    for c in range(2):
        hc = h[:, 64 * c:64 * c + 64]
        hs_ref[c] = hc
        acc_ref[c, :, 0:64] = hc * jnp.dot(ex_self[:, 4 * c:4 * c + 4], _bc4_64())
        acc_ref[c, :, 64:68] = den8[:, 4 * c:4 * c + 4]
        acc_ref[c, :, 68:80] = jnp.zeros((NB, 12), jnp.float32)


_tc1 = pl.pallas_call(
    _tc1_body,
    grid=(NBLK,),
    in_specs=[
        pl.BlockSpec((NB, D), lambda i: (i, 0)),
        pl.BlockSpec((D, D), lambda i: (0, 0)),
        pl.BlockSpec((1, D), lambda i: (0, 0)),
        pl.BlockSpec((1, D), lambda i: (0, 0)),
    ],
    out_specs=[
        pl.BlockSpec((2, NB, 64), lambda i: (0, i, 0)),
        pl.BlockSpec((NB, ROWA), lambda i: (i, 0)),
        pl.BlockSpec((NB, ROWA), lambda i: (i, 0)),
        pl.BlockSpec((2, NB, ROWACC), lambda i: (0, i, 0)),
    ],
    out_shape=[
        jax.ShapeDtypeStruct((2, NPAD, 64), jnp.float32),
        jax.ShapeDtypeStruct((NPAD, ROWA), jnp.float32),
        jax.ShapeDtypeStruct((NPAD, ROWA), jnp.float32),
        jax.ShapeDtypeStruct((2, NPAD, ROWACC), jnp.float32),
    ],
)


def _tc2_body(acc_ref, w2_ref, b1_ref, ats2_ref, atd2_ref,
              hs2_ref, as2_ref, ad2_ref, acc2_ref):
    i = pl.program_id(0)
    parts = []
    for c in range(2):
        num = acc_ref[c, :, 0:64]
        den = acc_ref[c, :, 64:68]
        parts.append(num / jnp.dot(den, _bc4_64()))
    out1 = jnp.concatenate(parts, axis=1) + b1_ref[...]
    hmid = jnp.where(out1 > 0, out1, jnp.exp(jnp.minimum(out1, 0.0)) - 1.0)
    h2 = jnp.dot(hmid, w2_ref[...], preferred_element_type=jnp.float32)
    a_s = jnp.dot(h2 * ats2_ref[...], jnp.ones((D, 1), jnp.float32))
    a_d = jnp.dot(h2 * atd2_ref[...], jnp.ones((D, 1), jnp.float32))
    rows = i * NB + lax.broadcasted_iota(jnp.int32, (NB, 1), 0)
    mask = rows < N
    as2_ref[...] = jnp.dot(jnp.where(mask, a_s, _NEG), _eye(1, 16))
    ad2_ref[...] = jnp.dot(jnp.where(mask, a_d, 0.0), _eye(1, 16))
    ex_self = jnp.exp(_leaky(a_s + a_d))
    den1 = jnp.where(mask, ex_self, 1.0)
    for c in range(2):
        h2c = h2[:, 64 * c:64 * c + 64]
        hs2_ref[c] = h2c
        acc2_ref[c, :, 0:64] = h2c * ex_self
        acc2_ref[c, :, 64:65] = den1
        acc2_ref[c, :, 65:80] = jnp.zeros((NB, 15), jnp.float32)


_tc2 = pl.pallas_call(
    _tc2_body,
    grid=(NBLK,),
    in_specs=[
        pl.BlockSpec((2, NB, ROWACC), lambda i: (0, i, 0)),
        pl.BlockSpec((D, D), lambda i: (0, 0)),
        pl.BlockSpec((1, D), lambda i: (0, 0)),
        pl.BlockSpec((1, D), lambda i: (0, 0)),
        pl.BlockSpec((1, D), lambda i: (0, 0)),
    ],
    out_specs=[
        pl.BlockSpec((2, NB, 64), lambda i: (0, i, 0)),
        pl.BlockSpec((NB, ROWA), lambda i: (i, 0)),
        pl.BlockSpec((NB, ROWA), lambda i: (i, 0)),
        pl.BlockSpec((2, NB, ROWACC), lambda i: (0, i, 0)),
    ],
    out_shape=[
        jax.ShapeDtypeStruct((2, NPAD, 64), jnp.float32),
        jax.ShapeDtypeStruct((NPAD, ROWA), jnp.float32),
        jax.ShapeDtypeStruct((NPAD, ROWA), jnp.float32),
        jax.ShapeDtypeStruct((2, NPAD, ROWACC), jnp.float32),
    ],
)


def _tc3_body(acc_ref, b2_ref, batch_ref, out_ref):
    i = pl.program_id(0)
    parts = []
    for c in range(2):
        parts.append(acc_ref[c, :, 0:64] / acc_ref[c, :, 64:65])
    out2 = jnp.concatenate(parts, axis=1) + b2_ref[...]
    oh = (batch_ref[...] == lax.broadcasted_iota(jnp.int32, (NB, G), 1))
    contrib = lax.dot_general(oh.astype(jnp.float32), out2,
                              (((0,), (0,)), ((), ())),
                              preferred_element_type=jnp.float32)

    @pl.when(i == 0)
    def _():
        out_ref[...] = contrib

    @pl.when(i > 0)
    def _():
        out_ref[...] += contrib


_tc3 = pl.pallas_call(
    _tc3_body,
    grid=(NBLK,),
    in_specs=[
        pl.BlockSpec((2, NB, ROWACC), lambda i: (0, i, 0)),
        pl.BlockSpec((1, D), lambda i: (0, 0)),
        pl.BlockSpec((NB, 1), lambda i: (i, 0)),
    ],
    out_specs=pl.BlockSpec((G, D), lambda i: (0, 0)),
    out_shape=jax.ShapeDtypeStruct((G, D), jnp.float32),
)


def _take16(v, idx):
    dnums = lax.GatherDimensionNumbers(
        offset_dims=(), collapsed_slice_dims=(0,), start_index_map=(0,))
    return lax.gather(v, idx[:, None], dnums, slice_sizes=(1,),
                      mode=lax.GatherScatterMode.PROMISE_IN_BOUNDS)


def _make_sc_edge(hpc):
    mesh = plsc.VectorSubcoreMesh(core_axis_name="c", subcore_axis_name="s")

    @functools.partial(
        pl.kernel,
        out_type=jax.ShapeDtypeStruct((CORES * NPAD, ROWACC), jnp.float32),
        mesh=mesh,
        compiler_params=pltpu.CompilerParams(use_tc_tiling_on_sc=False),
        scratch_types=[
            pltpu.VMEM((C,), jnp.int32),
            pltpu.VMEM((C,), jnp.int32),
            pltpu.VMEM((C,), jnp.int32),
            pltpu.VMEM((C, ROWA), jnp.float32),
            pltpu.VMEM((C, ROWA), jnp.float32),
            pltpu.VMEM((C, ROWH), jnp.float32),
            pltpu.VMEM((C, ROWACC), jnp.float32),
            pltpu.VMEM_SHARED((NPAD, ROWACC), jnp.float32),
        ],
    )
    def sc_edge(src_hbm, dst_hbm, astab_hbm, adtab_hbm, hsplit_hbm, accinit_hbm,
                out_hbm, srcv, dstv, hidxv, avs, avd, hv, mv, acc_sh):
        cid = lax.axis_index("c")
        sid = lax.axis_index("s")
        r0 = sid * RPS
        pltpu.sync_copy(accinit_hbm.at[pl.ds(cid * NPAD + r0, RPS)],
                        acc_sh.at[pl.ds(r0, RPS)])
        plsc.subcore_barrier()

        iota16 = lax.iota(jnp.int32, 16)
        lane0 = cid * 4 if hpc == 4 else 0
        tail_idx = jnp.minimum(iota16, hpc - 1) + lane0
        tail_mask = iota16 < hpc
        ebase = sid * EPW

        def chunk_body(i, _):
            off = ebase + i * C
            pltpu.sync_copy(src_hbm.at[pl.ds(off, C)], srcv)
            pltpu.sync_copy(dst_hbm.at[pl.ds(off, C)], dstv)
            for k in range(C // 16):
                hidxv[pl.ds(16 * k, 16)] = srcv[pl.ds(16 * k, 16)] + cid * NPAD
            pltpu.sync_copy(astab_hbm.at[srcv], avs)
            pltpu.sync_copy(adtab_hbm.at[dstv], avd)
            pltpu.sync_copy(hsplit_hbm.at[hidxv], hv)

            def edge_body(e, _):
                ex = jnp.exp(_leaky(avs[e] + avd[e]))
                if hpc == 1:
                    exb0 = _take16(ex, jnp.zeros((16,), jnp.int32))
                for j in range(4):
                    if hpc == 4:
                        exb = _take16(ex, jnp.full((16,), j, jnp.int32) + lane0)
                    else:
                        exb = exb0
                    mv[e, pl.ds(16 * j, 16)] = hv[e, pl.ds(16 * j, 16)] * exb
                tail = jnp.where(tail_mask, _take16(ex, tail_idx), 0.0)
                mv[e, pl.ds(64, 16)] = tail
                return 0

            lax.fori_loop(0, C, edge_body, 0)
            pltpu.sync_copy(mv, acc_sh.at[dstv], add=True)
            return 0

        lax.fori_loop(0, NCHUNK, chunk_body, 0)
        plsc.subcore_barrier()
        pltpu.sync_copy(acc_sh.at[pl.ds(r0, RPS)],
                        out_hbm.at[pl.ds(cid * NPAD + r0, RPS)])

    return sc_edge


_sc_edge_l1 = _make_sc_edge(4)
_sc_edge_l2 = _make_sc_edge(1)


def kernel(x, edge_index, batch, W1, att_src1, att_dst1, b1,
           W2, att_src2, att_dst2, b2):
    x_p = jnp.pad(x, ((0, NPAD - N), (0, 0)))
    pad_idx = N + (jnp.arange(EPAD - E, dtype=jnp.int32) % (NPAD - N))
    src = jnp.concatenate([edge_index[0], pad_idx])
    dst = jnp.concatenate([edge_index[1], pad_idx])
    batch_p = jnp.pad(batch, (0, NPAD - N), constant_values=G).reshape(NPAD, 1)

    hs, as1, ad1, acc1_init = _tc1(x_p, W1,
                                   att_src1.reshape(1, D),
                                   att_dst1.reshape(1, D))
    acc1 = _sc_edge_l1(src, dst, as1, ad1,
                       hs.reshape(CORES * NPAD, ROWH),
                       acc1_init.reshape(CORES * NPAD, ROWACC))
    hs2, as2, ad2, acc2_init = _tc2(acc1.reshape(CORES, NPAD, ROWACC), W2,
                                    b1.reshape(1, D),
                                    att_src2.reshape(1, D),
                                    att_dst2.reshape(1, D))
    acc2 = _sc_edge_l2(src, dst, as2, ad2,
                       hs2.reshape(CORES * NPAD, ROWH),
                       acc2_init.reshape(CORES * NPAD, ROWACC))
    return _tc3(acc2.reshape(CORES, NPAD, ROWACC), b2.reshape(1, D), batch_p)


# speedup vs baseline: 103.7626x; 5.0250x over previous
import functools

import numpy as np
import jax
import jax.numpy as jnp
from jax import lax
from jax.experimental import pallas as pl
from jax.experimental.pallas import tpu as pltpu
from jax.experimental.pallas import tpu_sc as plsc

N = 10000
E = 320000
G = 64
D = 128

NPAD = 10240
NB = 1024
NBLK = NPAD // NB

CORES = 2
SUBC = 16
C = 128
EPW = 20224
EPAD = EPW * SUBC
NCHUNK = EPW // C
NG = NCHUNK // 2
RPS = NPAD // SUBC

ROWA = 16
ROWH = 64
ROWACC = 80

_NEG = -1e30

def _kred8():
    r = lax.broadcasted_iota(jnp.int32, (D, 8), 0)
    c = lax.broadcasted_iota(jnp.int32, (D, 8), 1)
    return (r // 16 == c).astype(jnp.float32)


def _eye(m, n):
    r = lax.broadcasted_iota(jnp.int32, (m, n), 0)
    c = lax.broadcasted_iota(jnp.int32, (m, n), 1)
    return (r == c).astype(jnp.float32)


def _bc4_64():
    r = lax.broadcasted_iota(jnp.int32, (4, 64), 0)
    c = lax.broadcasted_iota(jnp.int32, (4, 64), 1)
    return (c // 16 == r).astype(jnp.float32)


def _leaky(x):
    return jnp.where(x >= 0, x, x * jnp.float32(0.2))


def _tc1_body(x_ref, w_ref, ats_ref, atd_ref,
              hs_ref, as_ref, ad_ref, acc_ref):
    i = pl.program_id(0)
    h = jnp.dot(x_ref[...], w_ref[...], preferred_element_type=jnp.float32)
    a_s = jnp.dot(h * ats_ref[...], _kred8())
    a_d = jnp.dot(h * atd_ref[...], _kred8())
    rows = i * NB + lax.broadcasted_iota(jnp.int32, (NB, 1), 0)
    mask = rows < N
    as_ref[...] = jnp.dot(jnp.where(mask, a_s, _NEG), _eye(8, 16))
    ad_ref[...] = jnp.dot(jnp.where(mask, a_d, 0.0), _eye(8, 16))
    ex_self = jnp.exp(_leaky(a_s + a_d))
    den8 = jnp.where(mask, ex_self, 1.0)
    for c in range(2):
        hc = h[:, 64 * c:64 * c + 64]
        hs_ref[c] = hc
        acc_ref[c, :, 0:64] = hc * jnp.dot(ex_self[:, 4 * c:4 * c + 4], _bc4_64())
        acc_ref[c, :, 64:68] = den8[:, 4 * c:4 * c + 4]
        acc_ref[c, :, 68:80] = jnp.zeros((NB, 12), jnp.float32)


_tc1 = pl.pallas_call(
    _tc1_body,
    grid=(NBLK,),
    in_specs=[
        pl.BlockSpec((NB, D), lambda i: (i, 0)),
        pl.BlockSpec((D, D), lambda i: (0, 0)),
        pl.BlockSpec((1, D), lambda i: (0, 0)),
        pl.BlockSpec((1, D), lambda i: (0, 0)),
    ],
    out_specs=[
        pl.BlockSpec((2, NB, 64), lambda i: (0, i, 0)),
        pl.BlockSpec((NB, ROWA), lambda i: (i, 0)),
        pl.BlockSpec((NB, ROWA), lambda i: (i, 0)),
        pl.BlockSpec((2, NB, ROWACC), lambda i: (0, i, 0)),
    ],
    out_shape=[
        jax.ShapeDtypeStruct((2, NPAD, 64), jnp.float32),
        jax.ShapeDtypeStruct((NPAD, ROWA), jnp.float32),
        jax.ShapeDtypeStruct((NPAD, ROWA), jnp.float32),
        jax.ShapeDtypeStruct((2, NPAD, ROWACC), jnp.float32),
    ],
)


def _tc2_body(acc_ref, w2_ref, b1_ref, ats2_ref, atd2_ref,
              hs2_ref, as2_ref, ad2_ref, acc2_ref):
    i = pl.program_id(0)
    parts = []
    for c in range(2):
        num = acc_ref[c, :, 0:64]
        den = acc_ref[c, :, 64:68]
        parts.append(num / jnp.dot(den, _bc4_64()))
    out1 = jnp.concatenate(parts, axis=1) + b1_ref[...]
    hmid = jnp.where(out1 > 0, out1, jnp.exp(jnp.minimum(out1, 0.0)) - 1.0)
    h2 = jnp.dot(hmid, w2_ref[...], preferred_element_type=jnp.float32)
    a_s = jnp.dot(h2 * ats2_ref[...], jnp.ones((D, 1), jnp.float32))
    a_d = jnp.dot(h2 * atd2_ref[...], jnp.ones((D, 1), jnp.float32))
    rows = i * NB + lax.broadcasted_iota(jnp.int32, (NB, 1), 0)
    mask = rows < N
    as2_ref[...] = jnp.dot(jnp.where(mask, a_s, _NEG), _eye(1, 16))
    ad2_ref[...] = jnp.dot(jnp.where(mask, a_d, 0.0), _eye(1, 16))
    ex_self = jnp.exp(_leaky(a_s + a_d))
    den1 = jnp.where(mask, ex_self, 1.0)
    for c in range(2):
        h2c = h2[:, 64 * c:64 * c + 64]
        hs2_ref[c] = h2c
        acc2_ref[c, :, 0:64] = h2c * ex_self
        acc2_ref[c, :, 64:65] = den1
        acc2_ref[c, :, 65:80] = jnp.zeros((NB, 15), jnp.float32)


_tc2 = pl.pallas_call(
    _tc2_body,
    grid=(NBLK,),
    in_specs=[
        pl.BlockSpec((2, NB, ROWACC), lambda i: (0, i, 0)),
        pl.BlockSpec((D, D), lambda i: (0, 0)),
        pl.BlockSpec((1, D), lambda i: (0, 0)),
        pl.BlockSpec((1, D), lambda i: (0, 0)),
        pl.BlockSpec((1, D), lambda i: (0, 0)),
    ],
    out_specs=[
        pl.BlockSpec((2, NB, 64), lambda i: (0, i, 0)),
        pl.BlockSpec((NB, ROWA), lambda i: (i, 0)),
        pl.BlockSpec((NB, ROWA), lambda i: (i, 0)),
        pl.BlockSpec((2, NB, ROWACC), lambda i: (0, i, 0)),
    ],
    out_shape=[
        jax.ShapeDtypeStruct((2, NPAD, 64), jnp.float32),
        jax.ShapeDtypeStruct((NPAD, ROWA), jnp.float32),
        jax.ShapeDtypeStruct((NPAD, ROWA), jnp.float32),
        jax.ShapeDtypeStruct((2, NPAD, ROWACC), jnp.float32),
    ],
)


def _tc3_body(acc_ref, b2_ref, batch_ref, out_ref):
    i = pl.program_id(0)
    parts = []
    for c in range(2):
        parts.append(acc_ref[c, :, 0:64] / acc_ref[c, :, 64:65])
    out2 = jnp.concatenate(parts, axis=1) + b2_ref[...]
    oh = (batch_ref[...] == lax.broadcasted_iota(jnp.int32, (NB, G), 1))
    contrib = lax.dot_general(oh.astype(jnp.float32), out2,
                              (((0,), (0,)), ((), ())),
                              preferred_element_type=jnp.float32)

    @pl.when(i == 0)
    def _():
        out_ref[...] = contrib

    @pl.when(i > 0)
    def _():
        out_ref[...] += contrib


_tc3 = pl.pallas_call(
    _tc3_body,
    grid=(NBLK,),
    in_specs=[
        pl.BlockSpec((2, NB, ROWACC), lambda i: (0, i, 0)),
        pl.BlockSpec((1, D), lambda i: (0, 0)),
        pl.BlockSpec((NB, 1), lambda i: (i, 0)),
    ],
    out_specs=pl.BlockSpec((G, D), lambda i: (0, 0)),
    out_shape=jax.ShapeDtypeStruct((G, D), jnp.float32),
)


def _take16(v, idx):
    dnums = lax.GatherDimensionNumbers(
        offset_dims=(), collapsed_slice_dims=(0,), start_index_map=(0,))
    return lax.gather(v, idx[:, None], dnums, slice_sizes=(1,),
                      mode=lax.GatherScatterMode.PROMISE_IN_BOUNDS)


def _make_sc_edge(hpc):
    mesh = plsc.VectorSubcoreMesh(core_axis_name="c", subcore_axis_name="s")

    @functools.partial(
        pl.kernel,
        out_type=jax.ShapeDtypeStruct((CORES * NPAD, ROWACC), jnp.float32),
        mesh=mesh,
        compiler_params=pltpu.CompilerParams(use_tc_tiling_on_sc=False),
        scratch_types=[
            pltpu.VMEM((2, C), jnp.int32),
            pltpu.VMEM((2, C), jnp.int32),
            pltpu.VMEM((2, C), jnp.int32),
            pltpu.VMEM((2, C), jnp.int32),
            pltpu.VMEM((2, C, ROWA), jnp.float32),
            pltpu.VMEM((2, C, ROWA), jnp.float32),
            pltpu.VMEM((2, C, ROWH), jnp.float32),
            pltpu.VMEM((2, C, ROWACC), jnp.float32),
            pltpu.VMEM_SHARED((NPAD, ROWACC), jnp.float32),
            pltpu.SemaphoreType.DMA,
            pltpu.SemaphoreType.DMA,
            pltpu.SemaphoreType.DMA,
            pltpu.SemaphoreType.DMA,
            pltpu.SemaphoreType.DMA,
            pltpu.SemaphoreType.DMA,
        ],
    )
    def sc_edge(src_hbm, dst_hbm, astab_hbm, adtab_hbm, hsplit_hbm, accinit_hbm,
                out_hbm, srcv, dstv, hidxv, sdst, avs, avd, hv, mv, acc_sh,
                isem0, isem1, gsem0, gsem1, ssem0, ssem1):
        cid = lax.axis_index("c")
        sid = lax.axis_index("s")
        r0 = sid * RPS
        isem = (isem0, isem1)
        gsem = (gsem0, gsem1)
        ssem = (ssem0, ssem1)
        pltpu.sync_copy(accinit_hbm.at[pl.ds(cid * NPAD + r0, RPS)],
                        acc_sh.at[pl.ds(r0, RPS)])
        plsc.subcore_barrier()

        iota16 = lax.iota(jnp.int32, 16)
        lane0 = cid * 4 if hpc == 4 else 0
        tail_idx = jnp.minimum(iota16, hpc - 1) + lane0
        tail_mask = iota16 < hpc
        ebase = sid * EPW

        def idx_start(b, i):
            off = ebase + i * C
            pltpu.async_copy(src_hbm.at[pl.ds(off, C)], srcv.at[b], isem[b])
            pltpu.async_copy(dst_hbm.at[pl.ds(off, C)], dstv.at[b], isem[b])

        def idx_wait(b):
            pltpu.make_async_copy(src_hbm.at[pl.ds(0, C)], srcv.at[b], isem[b]).wait()
            pltpu.make_async_copy(dst_hbm.at[pl.ds(0, C)], dstv.at[b], isem[b]).wait()

        def hidx_compute(b):
            for k in range(C // 16):
                hidxv[b, pl.ds(16 * k, 16)] = (
                    srcv[b, pl.ds(16 * k, 16)] + cid * NPAD)

        def gather_start(b):
            pltpu.async_copy(astab_hbm.at[srcv.at[b]], avs.at[b], gsem[b])
            pltpu.async_copy(adtab_hbm.at[dstv.at[b]], avd.at[b], gsem[b])
            pltpu.async_copy(hsplit_hbm.at[hidxv.at[b]], hv.at[b], gsem[b])

        def gather_wait(b):
            pltpu.make_async_copy(astab_hbm.at[srcv.at[b]], avs.at[b], gsem[b]).wait()
            pltpu.make_async_copy(adtab_hbm.at[dstv.at[b]], avd.at[b], gsem[b]).wait()
            pltpu.make_async_copy(hsplit_hbm.at[hidxv.at[b]], hv.at[b], gsem[b]).wait()

        def scatter_start(b):
            pltpu.async_copy(mv.at[b], acc_sh.at[sdst.at[b]], ssem[b], add=True)

        def scatter_wait(b):
            pltpu.make_async_copy(mv.at[b], acc_sh.at[sdst.at[b]], ssem[b]).wait()

        def compute(b):
            avs_b, avd_b, hv_b, mv_b = avs.at[b], avd.at[b], hv.at[b], mv.at[b]

            @plsc.parallel_loop(0, C, unroll=4)
            def _(e):
                ex = jnp.exp(_leaky(avs_b[e] + avd_b[e]))
                if hpc == 1:
                    exb0 = _take16(ex, jnp.zeros((16,), jnp.int32))
                for j in range(4):
                    if hpc == 4:
                        exb = _take16(ex, jnp.full((16,), j, jnp.int32) + lane0)
                    else:
                        exb = exb0
                    mv_b[e, pl.ds(16 * j, 16)] = hv_b[e, pl.ds(16 * j, 16)] * exb
                tail = jnp.where(tail_mask, _take16(ex, tail_idx), 0.0)
                mv_b[e, pl.ds(64, 16)] = tail

        idx_start(0, 0)
        idx_start(1, 1)
        idx_wait(0)
        hidx_compute(0)
        gather_start(0)

        def pair_body(g, _):
            for b in range(2):
                i = 2 * g + b
                ob = 1 - b
                if b == 0:
                    idx_wait(ob)
                    hidx_compute(ob)
                    gather_start(ob)
                else:
                    @pl.when(g < NG - 1)
                    def _():
                        idx_wait(ob)
                        hidx_compute(ob)
                        gather_start(ob)
                gather_wait(b)

                @pl.when(g >= 1)
                def _():
                    scatter_wait(b)
                for k in range(C // 16):
                    sdst[b, pl.ds(16 * k, 16)] = dstv[b, pl.ds(16 * k, 16)]

                @pl.when(g < NG - 1)
                def _():
                    idx_start(b, i + 2)
                compute(b)
                scatter_start(b)
            return 0

        lax.fori_loop(0, NG, pair_body, 0)
        scatter_wait(0)
        scatter_wait(1)
        plsc.subcore_barrier()
        pltpu.sync_copy(acc_sh.at[pl.ds(r0, RPS)],
                        out_hbm.at[pl.ds(cid * NPAD + r0, RPS)])

    return sc_edge


_sc_edge_l1 = _make_sc_edge(4)
_sc_edge_l2 = _make_sc_edge(1)


def kernel(x, edge_index, batch, W1, att_src1, att_dst1, b1,
           W2, att_src2, att_dst2, b2):
    x_p = jnp.pad(x, ((0, NPAD - N), (0, 0)))
    pad_idx = N + (jnp.arange(EPAD - E, dtype=jnp.int32) % (NPAD - N))
    src = jnp.concatenate([edge_index[0], pad_idx])
    dst = jnp.concatenate([edge_index[1], pad_idx])
    batch_p = jnp.pad(batch, (0, NPAD - N), constant_values=G).reshape(NPAD, 1)

    hs, as1, ad1, acc1_init = _tc1(x_p, W1,
                                   att_src1.reshape(1, D),
                                   att_dst1.reshape(1, D))
    acc1 = _sc_edge_l1(src, dst, as1, ad1,
                       hs.reshape(CORES * NPAD, ROWH),
                       acc1_init.reshape(CORES * NPAD, ROWACC))
    hs2, as2, ad2, acc2_init = _tc2(acc1.reshape(CORES, NPAD, ROWACC), W2,
                                    b1.reshape(1, D),
                                    att_src2.reshape(1, D),
                                    att_dst2.reshape(1, D))
    acc2 = _sc_edge_l2(src, dst, as2, ad2,
                       hs2.reshape(CORES * NPAD, ROWH),
                       acc2_init.reshape(CORES * NPAD, ROWACC))
    return _tc3(acc2.reshape(CORES, NPAD, ROWACC), b2.reshape(1, D), batch_p)


# speedup vs baseline: 113.6383x; 1.0952x over previous
import functools

import numpy as np
import jax
import jax.numpy as jnp
from jax import lax
from jax.experimental import pallas as pl
from jax.experimental.pallas import tpu as pltpu
from jax.experimental.pallas import tpu_sc as plsc

N = 10000
E = 320000
G = 64
D = 128

NPAD = 10240
NB = 1024
NBLK = NPAD // NB

CORES = 2
SUBC = 16
C = 128
EPW = 20352
EPAD = EPW * SUBC
NCHUNK = EPW // C
NG = NCHUNK // 3
RPS = NPAD // SUBC

ROWA = 16
ROWH = 64
ROWACC = 80

_NEG = -1e30

def _kred8():
    r = lax.broadcasted_iota(jnp.int32, (D, 8), 0)
    c = lax.broadcasted_iota(jnp.int32, (D, 8), 1)
    return (r // 16 == c).astype(jnp.float32)


def _eye(m, n):
    r = lax.broadcasted_iota(jnp.int32, (m, n), 0)
    c = lax.broadcasted_iota(jnp.int32, (m, n), 1)
    return (r == c).astype(jnp.float32)


def _bc4_64():
    r = lax.broadcasted_iota(jnp.int32, (4, 64), 0)
    c = lax.broadcasted_iota(jnp.int32, (4, 64), 1)
    return (c // 16 == r).astype(jnp.float32)


def _tperm(p):
    return 32 * (p // 32) + 2 * (p % 16) + (p // 16) % 2


def _permP():
    r = lax.broadcasted_iota(jnp.int32, (64, 64), 0)
    c = lax.broadcasted_iota(jnp.int32, (64, 64), 1)
    return (c == _tperm(r)).astype(jnp.float32)


def _permPT():
    r = lax.broadcasted_iota(jnp.int32, (64, 64), 0)
    c = lax.broadcasted_iota(jnp.int32, (64, 64), 1)
    return (r == _tperm(c)).astype(jnp.float32)


def _bc4p():
    r = lax.broadcasted_iota(jnp.int32, (4, 64), 0)
    c = lax.broadcasted_iota(jnp.int32, (4, 64), 1)
    return (_tperm(c) // 16 == r).astype(jnp.float32)


def _sel_even():
    r = lax.broadcasted_iota(jnp.int32, (64, 32), 0)
    c = lax.broadcasted_iota(jnp.int32, (64, 32), 1)
    return (r == 2 * c).astype(jnp.float32)


def _sel_odd():
    r = lax.broadcasted_iota(jnp.int32, (64, 32), 0)
    c = lax.broadcasted_iota(jnp.int32, (64, 32), 1)
    return (r == 2 * c + 1).astype(jnp.float32)


def _pack_bf16_pair(even, odd):
    ue = lax.bitcast_convert_type(even, jnp.uint32)
    uo = lax.bitcast_convert_type(odd, jnp.uint32)
    ue = (ue + 0x7FFF + ((ue >> 16) & 1)) >> 16
    uo = (uo + 0x7FFF + ((uo >> 16) & 1)) >> 16
    return lax.bitcast_convert_type(ue | (uo << 16), jnp.int32)


def _leaky(x):
    return jnp.where(x >= 0, x, x * jnp.float32(0.2))


def _tc1_body(x_ref, w_ref, ats_ref, atd_ref,
              hs_ref, as_ref, ad_ref, acc_ref):
    i = pl.program_id(0)
    h = jnp.dot(x_ref[...], w_ref[...], preferred_element_type=jnp.float32)
    a_s = jnp.dot(h * ats_ref[...], _kred8())
    a_d = jnp.dot(h * atd_ref[...], _kred8())
    rows = i * NB + lax.broadcasted_iota(jnp.int32, (NB, 1), 0)
    mask = rows < N
    as_ref[...] = jnp.dot(jnp.where(mask, a_s, _NEG), _eye(8, 16))
    ad_ref[...] = jnp.dot(jnp.where(mask, a_d, 0.0), _eye(8, 16))
    ex_self = jnp.exp(_leaky(a_s + a_d))
    den8 = jnp.where(mask, ex_self, 1.0)
    for c in range(2):
        hc = h[:, 64 * c:64 * c + 64]
        hs_ref[c] = hc
        acc_ref[c, :, 0:64] = hc * jnp.dot(ex_self[:, 4 * c:4 * c + 4], _bc4_64())
        acc_ref[c, :, 64:68] = den8[:, 4 * c:4 * c + 4]
        acc_ref[c, :, 68:80] = jnp.zeros((NB, 12), jnp.float32)


_tc1 = pl.pallas_call(
    _tc1_body,
    grid=(NBLK,),
    in_specs=[
        pl.BlockSpec((NB, D), lambda i: (i, 0)),
        pl.BlockSpec((D, D), lambda i: (0, 0)),
        pl.BlockSpec((1, D), lambda i: (0, 0)),
        pl.BlockSpec((1, D), lambda i: (0, 0)),
    ],
    out_specs=[
        pl.BlockSpec((2, NB, 64), lambda i: (0, i, 0)),
        pl.BlockSpec((NB, ROWA), lambda i: (i, 0)),
        pl.BlockSpec((NB, ROWA), lambda i: (i, 0)),
        pl.BlockSpec((2, NB, ROWACC), lambda i: (0, i, 0)),
    ],
    out_shape=[
        jax.ShapeDtypeStruct((2, NPAD, 64), jnp.float32),
        jax.ShapeDtypeStruct((NPAD, ROWA), jnp.float32),
        jax.ShapeDtypeStruct((NPAD, ROWA), jnp.float32),
        jax.ShapeDtypeStruct((2, NPAD, ROWACC), jnp.float32),
    ],
)


def _tc2_body(acc_ref, w2_ref, b1_ref, ats2_ref, atd2_ref,
              hs2_ref, as2_ref, ad2_ref, acc2_ref):
    i = pl.program_id(0)
    parts = []
    for c in range(2):
        num = acc_ref[c, :, 0:64]
        den = acc_ref[c, :, 64:68]
        parts.append(num / jnp.dot(den, _bc4_64()))
    out1 = jnp.concatenate(parts, axis=1) + b1_ref[...]
    hmid = jnp.where(out1 > 0, out1, jnp.exp(jnp.minimum(out1, 0.0)) - 1.0)
    h2 = jnp.dot(hmid, w2_ref[...], preferred_element_type=jnp.float32)
    a_s = jnp.dot(h2 * ats2_ref[...], jnp.ones((D, 1), jnp.float32))
    a_d = jnp.dot(h2 * atd2_ref[...], jnp.ones((D, 1), jnp.float32))
    rows = i * NB + lax.broadcasted_iota(jnp.int32, (NB, 1), 0)
    mask = rows < N
    as2_ref[...] = jnp.dot(jnp.where(mask, a_s, _NEG), _eye(1, 16))
    ad2_ref[...] = jnp.dot(jnp.where(mask, a_d, 0.0), _eye(1, 16))
    ex_self = jnp.exp(_leaky(a_s + a_d))
    den1 = jnp.where(mask, ex_self, 1.0)
    for c in range(2):
        h2c = h2[:, 64 * c:64 * c + 64]
        hs2_ref[c] = h2c
        acc2_ref[c, :, 0:64] = h2c * ex_self
        acc2_ref[c, :, 64:65] = den1
        acc2_ref[c, :, 65:80] = jnp.zeros((NB, 15), jnp.float32)


_tc2 = pl.pallas_call(
    _tc2_body,
    grid=(NBLK,),
    in_specs=[
        pl.BlockSpec((2, NB, ROWACC), lambda i: (0, i, 0)),
        pl.BlockSpec((D, D), lambda i: (0, 0)),
        pl.BlockSpec((1, D), lambda i: (0, 0)),
        pl.BlockSpec((1, D), lambda i: (0, 0)),
        pl.BlockSpec((1, D), lambda i: (0, 0)),
    ],
    out_specs=[
        pl.BlockSpec((2, NB, 64), lambda i: (0, i, 0)),
        pl.BlockSpec((NB, ROWA), lambda i: (i, 0)),
        pl.BlockSpec((NB, ROWA), lambda i: (i, 0)),
        pl.BlockSpec((2, NB, ROWACC), lambda i: (0, i, 0)),
    ],
    out_shape=[
        jax.ShapeDtypeStruct((2, NPAD, 64), jnp.float32),
        jax.ShapeDtypeStruct((NPAD, ROWA), jnp.float32),
        jax.ShapeDtypeStruct((NPAD, ROWA), jnp.float32),
        jax.ShapeDtypeStruct((2, NPAD, ROWACC), jnp.float32),
    ],
)


def _tc3_body(acc_ref, b2_ref, batch_ref, out_ref):
    i = pl.program_id(0)
    parts = []
    for c in range(2):
        parts.append(acc_ref[c, :, 0:64] / acc_ref[c, :, 64:65])
    out2 = jnp.concatenate(parts, axis=1) + b2_ref[...]
    oh = (batch_ref[...] == lax.broadcasted_iota(jnp.int32, (NB, G), 1))
    contrib = lax.dot_general(oh.astype(jnp.float32), out2,
                              (((0,), (0,)), ((), ())),
                              preferred_element_type=jnp.float32)

    @pl.when(i == 0)
    def _():
        out_ref[...] = contrib

    @pl.when(i > 0)
    def _():
        out_ref[...] += contrib


_tc3 = pl.pallas_call(
    _tc3_body,
    grid=(NBLK,),
    in_specs=[
        pl.BlockSpec((2, NB, ROWACC), lambda i: (0, i, 0)),
        pl.BlockSpec((1, D), lambda i: (0, 0)),
        pl.BlockSpec((NB, 1), lambda i: (i, 0)),
    ],
    out_specs=pl.BlockSpec((G, D), lambda i: (0, 0)),
    out_shape=jax.ShapeDtypeStruct((G, D), jnp.float32),
)


def _take16(v, idx):
    dnums = lax.GatherDimensionNumbers(
        offset_dims=(), collapsed_slice_dims=(0,), start_index_map=(0,))
    return lax.gather(v, idx[:, None], dnums, slice_sizes=(1,),
                      mode=lax.GatherScatterMode.PROMISE_IN_BOUNDS)


def _make_sc_edge(hpc):
    mesh = plsc.VectorSubcoreMesh(core_axis_name="c", subcore_axis_name="s")

    @functools.partial(
        pl.kernel,
        out_type=jax.ShapeDtypeStruct((CORES * NPAD, ROWACC), jnp.float32),
        mesh=mesh,
        compiler_params=pltpu.CompilerParams(use_tc_tiling_on_sc=False),
        scratch_types=[
            pltpu.VMEM((3, C), jnp.int32),
            pltpu.VMEM((3, C), jnp.int32),
            pltpu.VMEM((3, C), jnp.int32),
            pltpu.VMEM((3, C), jnp.int32),
            pltpu.VMEM((3, C, ROWA), jnp.float32),
            pltpu.VMEM((3, C, ROWA), jnp.float32),
            pltpu.VMEM((3, C, ROWH), jnp.float32),
            pltpu.VMEM((3, C, ROWACC), jnp.float32),
            pltpu.VMEM_SHARED((NPAD, ROWACC), jnp.float32),
            pltpu.SemaphoreType.DMA,
            pltpu.SemaphoreType.DMA,
            pltpu.SemaphoreType.DMA,
            pltpu.SemaphoreType.DMA,
            pltpu.SemaphoreType.DMA,
            pltpu.SemaphoreType.DMA,
            pltpu.SemaphoreType.DMA,
            pltpu.SemaphoreType.DMA,
            pltpu.SemaphoreType.DMA,
        ],
    )
    def sc_edge(src_hbm, dst_hbm, astab_hbm, adtab_hbm, hsplit_hbm, accinit_hbm,
                out_hbm, srcv, dstv, hidxv, sdst, avs, avd, hv, mv, acc_sh,
                isem0, isem1, isem2, gsem0, gsem1, gsem2, ssem0, ssem1, ssem2):
        cid = lax.axis_index("c")
        sid = lax.axis_index("s")
        r0 = sid * RPS
        isem = (isem0, isem1, isem2)
        gsem = (gsem0, gsem1, gsem2)
        ssem = (ssem0, ssem1, ssem2)
        pltpu.sync_copy(accinit_hbm.at[pl.ds(cid * NPAD + r0, RPS)],
                        acc_sh.at[pl.ds(r0, RPS)])
        plsc.subcore_barrier()

        iota16 = lax.iota(jnp.int32, 16)
        lane0 = cid * 4 if hpc == 4 else 0
        tail_idx = jnp.minimum(iota16, hpc - 1) + lane0
        tail_mask = iota16 < hpc
        ebase = sid * EPW

        def idx_start(b, i):
            off = ebase + i * C
            pltpu.async_copy(src_hbm.at[pl.ds(off, C)], srcv.at[b], isem[b])
            pltpu.async_copy(dst_hbm.at[pl.ds(off, C)], dstv.at[b], isem[b])

        def idx_wait(b):
            pltpu.make_async_copy(src_hbm.at[pl.ds(0, C)], srcv.at[b], isem[b]).wait()
            pltpu.make_async_copy(dst_hbm.at[pl.ds(0, C)], dstv.at[b], isem[b]).wait()

        def hidx_compute(b):
            for k in range(C // 16):
                hidxv[b, pl.ds(16 * k, 16)] = (
                    srcv[b, pl.ds(16 * k, 16)] + cid * NPAD)

        def gather_start(b):
            pltpu.async_copy(astab_hbm.at[srcv.at[b]], avs.at[b], gsem[b])
            pltpu.async_copy(adtab_hbm.at[dstv.at[b]], avd.at[b], gsem[b])
            pltpu.async_copy(hsplit_hbm.at[hidxv.at[b]], hv.at[b], gsem[b])

        def gather_wait(b):
            pltpu.make_async_copy(astab_hbm.at[srcv.at[b]], avs.at[b], gsem[b]).wait()
            pltpu.make_async_copy(adtab_hbm.at[dstv.at[b]], avd.at[b], gsem[b]).wait()
            pltpu.make_async_copy(hsplit_hbm.at[hidxv.at[b]], hv.at[b], gsem[b]).wait()

        def scatter_start(b):
            pltpu.async_copy(mv.at[b], acc_sh.at[sdst.at[b]], ssem[b], add=True)

        def scatter_wait(b):
            pltpu.make_async_copy(mv.at[b], acc_sh.at[sdst.at[b]], ssem[b]).wait()

        def compute(b):
            avs_b, avd_b, hv_b, mv_b = avs.at[b], avd.at[b], hv.at[b], mv.at[b]

            @plsc.parallel_loop(0, C, unroll=4)
            def _(e):
                ex = jnp.exp(_leaky(avs_b[e] + avd_b[e]))
                if hpc == 1:
                    exb0 = _take16(ex, jnp.zeros((16,), jnp.int32))
                for j in range(4):
                    if hpc == 4:
                        exb = _take16(ex, jnp.full((16,), j, jnp.int32) + lane0)
                    else:
                        exb = exb0
                    mv_b[e, pl.ds(16 * j, 16)] = hv_b[e, pl.ds(16 * j, 16)] * exb
                tail = jnp.where(tail_mask, _take16(ex, tail_idx), 0.0)
                mv_b[e, pl.ds(64, 16)] = tail

        idx_start(0, 0)
        idx_start(1, 1)
        idx_start(2, 2)
        idx_wait(0)
        hidx_compute(0)
        gather_start(0)
        idx_wait(1)
        hidx_compute(1)
        gather_start(1)

        def triple_body(g, _):
            for b in range(3):
                i = 3 * g + b
                nb2 = (b + 2) % 3
                if b == 0:
                    idx_wait(nb2)
                    hidx_compute(nb2)
                    gather_start(nb2)
                else:
                    @pl.when(g < NG - 1)
                    def _():
                        idx_wait(nb2)
                        hidx_compute(nb2)
                        gather_start(nb2)
                gather_wait(b)

                @pl.when(g >= 1)
                def _():
                    scatter_wait(b)
                for k in range(C // 16):
                    sdst[b, pl.ds(16 * k, 16)] = dstv[b, pl.ds(16 * k, 16)]

                @pl.when(g < NG - 1)
                def _():
                    idx_start(b, i + 3)
                compute(b)
                scatter_start(b)
            return 0

        lax.fori_loop(0, NG, triple_body, 0)
        scatter_wait(0)
        scatter_wait(1)
        scatter_wait(2)
        plsc.subcore_barrier()
        pltpu.sync_copy(acc_sh.at[pl.ds(r0, RPS)],
                        out_hbm.at[pl.ds(cid * NPAD + r0, RPS)])

    return sc_edge


_sc_edge_l1 = _make_sc_edge(4)
_sc_edge_l2 = _make_sc_edge(1)


def kernel(x, edge_index, batch, W1, att_src1, att_dst1, b1,
           W2, att_src2, att_dst2, b2):
    x_p = jnp.pad(x, ((0, NPAD - N), (0, 0)))
    pad_idx = N + (jnp.arange(EPAD - E, dtype=jnp.int32) % (NPAD - N))
    src = jnp.concatenate([edge_index[0], pad_idx])
    dst = jnp.concatenate([edge_index[1], pad_idx])
    batch_p = jnp.pad(batch, (0, NPAD - N), constant_values=G).reshape(NPAD, 1)

    hs, as1, ad1, acc1_init = _tc1(x_p, W1,
                                   att_src1.reshape(1, D),
                                   att_dst1.reshape(1, D))
    acc1 = _sc_edge_l1(src, dst, as1, ad1,
                       hs.reshape(CORES * NPAD, ROWH),
                       acc1_init.reshape(CORES * NPAD, ROWACC))
    hs2, as2, ad2, acc2_init = _tc2(acc1.reshape(CORES, NPAD, ROWACC), W2,
                                    b1.reshape(1, D),
                                    att_src2.reshape(1, D),
                                    att_dst2.reshape(1, D))
    acc2 = _sc_edge_l2(src, dst, as2, ad2,
                       hs2.reshape(CORES * NPAD, ROWH),
                       acc2_init.reshape(CORES * NPAD, ROWACC))
    return _tc3(acc2.reshape(CORES, NPAD, ROWACC), b2.reshape(1, D), batch_p)


# speedup vs baseline: 114.6658x; 1.0090x over previous
import functools

import numpy as np
import jax
import jax.numpy as jnp
from jax import lax
from jax.experimental import pallas as pl
from jax.experimental.pallas import tpu as pltpu
from jax.experimental.pallas import tpu_sc as plsc

N = 10000
E = 320000
G = 64
D = 128

NPAD = 10240
NB = 1024
NBLK = NPAD // NB

CORES = 2
SUBC = 16
C = 128
EPW = 20352
EPAD = EPW * SUBC
NCHUNK = EPW // C
NG = NCHUNK // 3
RPS = NPAD // SUBC

ROWA = 16
ROWH = 64
ROWACC = 80

_NEG = -1e30

def _kred8():
    r = lax.broadcasted_iota(jnp.int32, (D, 8), 0)
    c = lax.broadcasted_iota(jnp.int32, (D, 8), 1)
    return (r // 16 == c).astype(jnp.float32)


def _eye(m, n):
    r = lax.broadcasted_iota(jnp.int32, (m, n), 0)
    c = lax.broadcasted_iota(jnp.int32, (m, n), 1)
    return (r == c).astype(jnp.float32)


def _bc4_64():
    r = lax.broadcasted_iota(jnp.int32, (4, 64), 0)
    c = lax.broadcasted_iota(jnp.int32, (4, 64), 1)
    return (c // 16 == r).astype(jnp.float32)


def _tperm(p):
    return 32 * (p // 32) + 2 * (p % 16) + (p // 16) % 2


def _permP():
    r = lax.broadcasted_iota(jnp.int32, (64, 64), 0)
    c = lax.broadcasted_iota(jnp.int32, (64, 64), 1)
    return (c == _tperm(r)).astype(jnp.float32)


def _permPT():
    r = lax.broadcasted_iota(jnp.int32, (64, 64), 0)
    c = lax.broadcasted_iota(jnp.int32, (64, 64), 1)
    return (r == _tperm(c)).astype(jnp.float32)


def _bc4p():
    r = lax.broadcasted_iota(jnp.int32, (4, 64), 0)
    c = lax.broadcasted_iota(jnp.int32, (4, 64), 1)
    return (_tperm(c) // 16 == r).astype(jnp.float32)


def _sel_even():
    r = lax.broadcasted_iota(jnp.int32, (64, 32), 0)
    c = lax.broadcasted_iota(jnp.int32, (64, 32), 1)
    return (r == 2 * c).astype(jnp.float32)


def _sel_odd():
    r = lax.broadcasted_iota(jnp.int32, (64, 32), 0)
    c = lax.broadcasted_iota(jnp.int32, (64, 32), 1)
    return (r == 2 * c + 1).astype(jnp.float32)


def _pack_bf16_pair(even, odd):
    ue = lax.bitcast_convert_type(even, jnp.uint32)
    uo = lax.bitcast_convert_type(odd, jnp.uint32)
    ue = (ue + 0x7FFF + ((ue >> 16) & 1)) >> 16
    uo = (uo + 0x7FFF + ((uo >> 16) & 1)) >> 16
    return lax.bitcast_convert_type(ue | (uo << 16), jnp.int32)


def _leaky(x):
    return jnp.where(x >= 0, x, x * jnp.float32(0.2))


def _tc1_body(x_ref, w_ref, ats_ref, atd_ref,
              hs_ref, as_ref, ad_ref, exs_ref):
    i = pl.program_id(0)
    h = jnp.dot(x_ref[...], w_ref[...], preferred_element_type=jnp.float32)
    a_s = jnp.dot(h * ats_ref[...], _kred8())
    a_d = jnp.dot(h * atd_ref[...], _kred8())
    rows = i * NB + lax.broadcasted_iota(jnp.int32, (NB, 1), 0)
    mask = rows < N
    as_ref[...] = jnp.dot(jnp.where(mask, a_s, _NEG), _eye(8, 16))
    ad_ref[...] = jnp.dot(jnp.where(mask, a_d, 0.0), _eye(8, 16))
    ex_self = jnp.exp(_leaky(a_s + a_d))
    exs_ref[...] = jnp.dot(jnp.where(mask, ex_self, 1.0), _eye(8, 16))
    for c in range(2):
        hs_ref[c] = h[:, 64 * c:64 * c + 64]


_tc1 = pl.pallas_call(
    _tc1_body,
    grid=(NBLK,),
    in_specs=[
        pl.BlockSpec((NB, D), lambda i: (i, 0)),
        pl.BlockSpec((D, D), lambda i: (0, 0)),
        pl.BlockSpec((1, D), lambda i: (0, 0)),
        pl.BlockSpec((1, D), lambda i: (0, 0)),
    ],
    out_specs=[
        pl.BlockSpec((2, NB, 64), lambda i: (0, i, 0)),
        pl.BlockSpec((NB, ROWA), lambda i: (i, 0)),
        pl.BlockSpec((NB, ROWA), lambda i: (i, 0)),
        pl.BlockSpec((NB, ROWA), lambda i: (i, 0)),
    ],
    out_shape=[
        jax.ShapeDtypeStruct((2, NPAD, 64), jnp.float32),
        jax.ShapeDtypeStruct((NPAD, ROWA), jnp.float32),
        jax.ShapeDtypeStruct((NPAD, ROWA), jnp.float32),
        jax.ShapeDtypeStruct((NPAD, ROWA), jnp.float32),
    ],
)


def _tc2_body(acc_ref, w2_ref, b1_ref, ats2_ref, atd2_ref,
              hs2_ref, as2_ref, ad2_ref, exs2_ref):
    i = pl.program_id(0)
    parts = []
    for c in range(2):
        num = acc_ref[c, :, 0:64]
        den = acc_ref[c, :, 64:68]
        parts.append(num / jnp.dot(den, _bc4_64()))
    out1 = jnp.concatenate(parts, axis=1) + b1_ref[...]
    hmid = jnp.where(out1 > 0, out1, jnp.exp(jnp.minimum(out1, 0.0)) - 1.0)
    h2 = jnp.dot(hmid, w2_ref[...], preferred_element_type=jnp.float32)
    a_s = jnp.dot(h2 * ats2_ref[...], jnp.ones((D, 1), jnp.float32))
    a_d = jnp.dot(h2 * atd2_ref[...], jnp.ones((D, 1), jnp.float32))
    rows = i * NB + lax.broadcasted_iota(jnp.int32, (NB, 1), 0)
    mask = rows < N
    as2_ref[...] = jnp.dot(jnp.where(mask, a_s, _NEG), _eye(1, 16))
    ad2_ref[...] = jnp.dot(jnp.where(mask, a_d, 0.0), _eye(1, 16))
    ex_self = jnp.exp(_leaky(a_s + a_d))
    exs2_ref[...] = jnp.dot(jnp.where(mask, ex_self, 1.0), _eye(1, 16))
    for c in range(2):
        hs2_ref[c] = h2[:, 64 * c:64 * c + 64]


_tc2 = pl.pallas_call(
    _tc2_body,
    grid=(NBLK,),
    in_specs=[
        pl.BlockSpec((2, NB, ROWACC), lambda i: (0, i, 0)),
        pl.BlockSpec((D, D), lambda i: (0, 0)),
        pl.BlockSpec((1, D), lambda i: (0, 0)),
        pl.BlockSpec((1, D), lambda i: (0, 0)),
        pl.BlockSpec((1, D), lambda i: (0, 0)),
    ],
    out_specs=[
        pl.BlockSpec((2, NB, 64), lambda i: (0, i, 0)),
        pl.BlockSpec((NB, ROWA), lambda i: (i, 0)),
        pl.BlockSpec((NB, ROWA), lambda i: (i, 0)),
        pl.BlockSpec((NB, ROWA), lambda i: (i, 0)),
    ],
    out_shape=[
        jax.ShapeDtypeStruct((2, NPAD, 64), jnp.float32),
        jax.ShapeDtypeStruct((NPAD, ROWA), jnp.float32),
        jax.ShapeDtypeStruct((NPAD, ROWA), jnp.float32),
        jax.ShapeDtypeStruct((NPAD, ROWA), jnp.float32),
    ],
)


def _tc3_body(acc_ref, b2_ref, batch_ref, out_ref):
    i = pl.program_id(0)
    parts = []
    for c in range(2):
        parts.append(acc_ref[c, :, 0:64] / acc_ref[c, :, 64:65])
    out2 = jnp.concatenate(parts, axis=1) + b2_ref[...]
    oh = (batch_ref[...] == lax.broadcasted_iota(jnp.int32, (NB, G), 1))
    contrib = lax.dot_general(oh.astype(jnp.float32), out2,
                              (((0,), (0,)), ((), ())),
                              preferred_element_type=jnp.float32)

    @pl.when(i == 0)
    def _():
        out_ref[...] = contrib

    @pl.when(i > 0)
    def _():
        out_ref[...] += contrib


_tc3 = pl.pallas_call(
    _tc3_body,
    grid=(NBLK,),
    in_specs=[
        pl.BlockSpec((2, NB, ROWACC), lambda i: (0, i, 0)),
        pl.BlockSpec((1, D), lambda i: (0, 0)),
        pl.BlockSpec((NB, 1), lambda i: (i, 0)),
    ],
    out_specs=pl.BlockSpec((G, D), lambda i: (0, 0)),
    out_shape=jax.ShapeDtypeStruct((G, D), jnp.float32),
)


def _take16(v, idx):
    dnums = lax.GatherDimensionNumbers(
        offset_dims=(), collapsed_slice_dims=(0,), start_index_map=(0,))
    return lax.gather(v, idx[:, None], dnums, slice_sizes=(1,),
                      mode=lax.GatherScatterMode.PROMISE_IN_BOUNDS)


def _make_sc_edge(hpc):
    mesh = plsc.VectorSubcoreMesh(core_axis_name="c", subcore_axis_name="s")

    @functools.partial(
        pl.kernel,
        out_type=jax.ShapeDtypeStruct((CORES * NPAD, ROWACC), jnp.float32),
        mesh=mesh,
        compiler_params=pltpu.CompilerParams(use_tc_tiling_on_sc=False),
        scratch_types=[
            pltpu.VMEM((3, C), jnp.int32),
            pltpu.VMEM((3, C), jnp.int32),
            pltpu.VMEM((3, C), jnp.int32),
            pltpu.VMEM((3, C), jnp.int32),
            pltpu.VMEM((3, C, ROWA), jnp.float32),
            pltpu.VMEM((3, C, ROWA), jnp.float32),
            pltpu.VMEM((3, C, ROWH), jnp.float32),
            pltpu.VMEM((3, C, ROWACC), jnp.float32),
            pltpu.VMEM_SHARED((NPAD, ROWACC), jnp.float32),
            pltpu.SemaphoreType.DMA,
            pltpu.SemaphoreType.DMA,
            pltpu.SemaphoreType.DMA,
            pltpu.SemaphoreType.DMA,
            pltpu.SemaphoreType.DMA,
            pltpu.SemaphoreType.DMA,
            pltpu.SemaphoreType.DMA,
            pltpu.SemaphoreType.DMA,
            pltpu.SemaphoreType.DMA,
        ],
    )
    def sc_edge(src_hbm, dst_hbm, astab_hbm, adtab_hbm, hsplit_hbm, exstab_hbm,
                out_hbm, srcv, dstv, hidxv, sdst, avs, avd, hv, mv, acc_sh,
                isem0, isem1, isem2, gsem0, gsem1, gsem2, ssem0, ssem1, ssem2):
        cid = lax.axis_index("c")
        sid = lax.axis_index("s")
        r0 = sid * RPS
        isem = (isem0, isem1, isem2)
        gsem = (gsem0, gsem1, gsem2)
        ssem = (ssem0, ssem1, ssem2)

        iota16 = lax.iota(jnp.int32, 16)
        lane0 = cid * 4 if hpc == 4 else 0
        tail_idx = jnp.minimum(iota16, hpc - 1) + lane0
        tail_mask = iota16 < hpc
        ebase = sid * EPW

        for w in range(RPS // C):
            rr = r0 + w * C
            pltpu.sync_copy(hsplit_hbm.at[pl.ds(cid * NPAD + rr, C)], hv.at[0])
            pltpu.sync_copy(exstab_hbm.at[pl.ds(rr, C)], avs.at[0])
            exs_b, hvi_b, mvi_b = avs.at[0], hv.at[0], mv.at[0]

            @plsc.parallel_loop(0, C, unroll=4)
            def _(e):
                ex = exs_b[e]
                if hpc == 1:
                    exb0 = _take16(ex, jnp.zeros((16,), jnp.int32))
                for j in range(4):
                    if hpc == 4:
                        exb = _take16(ex, jnp.full((16,), j, jnp.int32) + lane0)
                    else:
                        exb = exb0
                    mvi_b[e, pl.ds(16 * j, 16)] = hvi_b[e, pl.ds(16 * j, 16)] * exb
                tail = jnp.where(tail_mask, _take16(ex, tail_idx), 0.0)
                mvi_b[e, pl.ds(64, 16)] = tail

            pltpu.sync_copy(mv.at[0], acc_sh.at[pl.ds(rr, C)])
        plsc.subcore_barrier()

        def idx_start(b, i):
            off = ebase + i * C
            pltpu.async_copy(src_hbm.at[pl.ds(off, C)], srcv.at[b], isem[b])
            pltpu.async_copy(dst_hbm.at[pl.ds(off, C)], dstv.at[b], isem[b])

        def idx_wait(b):
            pltpu.make_async_copy(src_hbm.at[pl.ds(0, C)], srcv.at[b], isem[b]).wait()
            pltpu.make_async_copy(dst_hbm.at[pl.ds(0, C)], dstv.at[b], isem[b]).wait()

        def hidx_compute(b):
            for k in range(C // 16):
                hidxv[b, pl.ds(16 * k, 16)] = (
                    srcv[b, pl.ds(16 * k, 16)] + cid * NPAD)

        def gather_start(b):
            pltpu.async_copy(astab_hbm.at[srcv.at[b]], avs.at[b], gsem[b])
            pltpu.async_copy(adtab_hbm.at[dstv.at[b]], avd.at[b], gsem[b])
            pltpu.async_copy(hsplit_hbm.at[hidxv.at[b]], hv.at[b], gsem[b])

        def gather_wait(b):
            pltpu.make_async_copy(astab_hbm.at[srcv.at[b]], avs.at[b], gsem[b]).wait()
            pltpu.make_async_copy(adtab_hbm.at[dstv.at[b]], avd.at[b], gsem[b]).wait()
            pltpu.make_async_copy(hsplit_hbm.at[hidxv.at[b]], hv.at[b], gsem[b]).wait()

        def scatter_start(b):
            pltpu.async_copy(mv.at[b], acc_sh.at[sdst.at[b]], ssem[b], add=True)

        def scatter_wait(b):
            pltpu.make_async_copy(mv.at[b], acc_sh.at[sdst.at[b]], ssem[b]).wait()

        def compute(b):
            avs_b, avd_b, hv_b, mv_b = avs.at[b], avd.at[b], hv.at[b], mv.at[b]

            @plsc.parallel_loop(0, C, unroll=4)
            def _(e):
                ex = jnp.exp(_leaky(avs_b[e] + avd_b[e]))
                if hpc == 1:
                    exb0 = _take16(ex, jnp.zeros((16,), jnp.int32))
                for j in range(4):
                    if hpc == 4:
                        exb = _take16(ex, jnp.full((16,), j, jnp.int32) + lane0)
                    else:
                        exb = exb0
                    mv_b[e, pl.ds(16 * j, 16)] = hv_b[e, pl.ds(16 * j, 16)] * exb
                tail = jnp.where(tail_mask, _take16(ex, tail_idx), 0.0)
                mv_b[e, pl.ds(64, 16)] = tail

        idx_start(0, 0)
        idx_start(1, 1)
        idx_start(2, 2)
        idx_wait(0)
        hidx_compute(0)
        gather_start(0)
        idx_wait(1)
        hidx_compute(1)
        gather_start(1)

        def triple_body(g, _):
            for b in range(3):
                i = 3 * g + b
                nb2 = (b + 2) % 3
                if b == 0:
                    idx_wait(nb2)
                    hidx_compute(nb2)
                    gather_start(nb2)
                else:
                    @pl.when(g < NG - 1)
                    def _():
                        idx_wait(nb2)
                        hidx_compute(nb2)
                        gather_start(nb2)
                gather_wait(b)

                @pl.when(g >= 1)
                def _():
                    scatter_wait(b)
                for k in range(C // 16):
                    sdst[b, pl.ds(16 * k, 16)] = dstv[b, pl.ds(16 * k, 16)]

                @pl.when(g < NG - 1)
                def _():
                    idx_start(b, i + 3)
                compute(b)
                scatter_start(b)
            return 0

        lax.fori_loop(0, NG, triple_body, 0)
        scatter_wait(0)
        scatter_wait(1)
        scatter_wait(2)
        plsc.subcore_barrier()
        pltpu.sync_copy(acc_sh.at[pl.ds(r0, RPS)],
                        out_hbm.at[pl.ds(cid * NPAD + r0, RPS)])

    return sc_edge


_sc_edge_l1 = _make_sc_edge(4)
_sc_edge_l2 = _make_sc_edge(1)


def kernel(x, edge_index, batch, W1, att_src1, att_dst1, b1,
           W2, att_src2, att_dst2, b2):
    x_p = jnp.pad(x, ((0, NPAD - N), (0, 0)))
    pad_idx = N + (jnp.arange(EPAD - E, dtype=jnp.int32) % (NPAD - N))
    src = jnp.concatenate([edge_index[0], pad_idx])
    dst = jnp.concatenate([edge_index[1], pad_idx])
    batch_p = jnp.pad(batch, (0, NPAD - N), constant_values=G).reshape(NPAD, 1)

    hs, as1, ad1, exs1 = _tc1(x_p, W1,
                              att_src1.reshape(1, D),
                              att_dst1.reshape(1, D))
    acc1 = _sc_edge_l1(src, dst, as1, ad1,
                       hs.reshape(CORES * NPAD, ROWH), exs1)
    hs2, as2, ad2, exs2 = _tc2(acc1.reshape(CORES, NPAD, ROWACC), W2,
                               b1.reshape(1, D),
                               att_src2.reshape(1, D),
                               att_dst2.reshape(1, D))
    acc2 = _sc_edge_l2(src, dst, as2, ad2,
                       hs2.reshape(CORES * NPAD, ROWH), exs2)
    return _tc3(acc2.reshape(CORES, NPAD, ROWACC), b2.reshape(1, D), batch_p)
